# Initial kernel scaffold; baseline (speedup 1.0000x reference)
import jax
import jax.numpy as jnp
from jax.experimental import pallas as pl


def kernel(logits):
    raise NotImplementedError("write your pallas kernel here")


# speedup vs baseline: 1.3160x; 1.3160x over previous
import functools

import jax
import jax.numpy as jnp
from jax.experimental import pallas as pl

_ROWS = 8
_V = 32768
_NCAND = 128


def _st_softmax_block(x_ref, o_ref):
    x = x_ref[...]
    m = jnp.max(x, axis=1, keepdims=True)
    u = jnp.exp(x - m)
    s = jnp.sum(u, axis=1, keepdims=True)
    umax = jnp.max(u, axis=1, keepdims=True)
    pmax = umax / s

    k = jax.lax.broadcasted_iota(jnp.int32, (_ROWS, _NCAND), 1)
    ucand = jax.lax.bitcast_convert_type(
        jax.lax.bitcast_convert_type(umax, jnp.int32) - k, jnp.float32)
    in_bucket = (ucand / s) == pmax
    c = jnp.min(jnp.where(in_bucket, ucand, jnp.inf), axis=1, keepdims=True)

    iota = jax.lax.broadcasted_iota(jnp.int32, (_ROWS, _V), 1)
    big = jnp.int32(2**31 - 1)
    idx = jnp.min(jnp.where(u >= c, iota, big), axis=1, keepdims=True)

    v = (1.0 - pmax) + pmax
    o_ref[...] = jnp.where(iota == idx, v, 0.0)


@jax.jit
def kernel(logits):
    b, h, vocab = logits.shape
    x = logits.reshape(b * h, vocab)
    out = pl.pallas_call(
        _st_softmax_block,
        grid=(b * h // _ROWS,),
        in_specs=[pl.BlockSpec((_ROWS, vocab), lambda i: (i, 0))],
        out_specs=pl.BlockSpec((_ROWS, vocab), lambda i: (i, 0)),
        out_shape=jax.ShapeDtypeStruct((b * h, vocab), jnp.float32),
    )(x)
    return out.reshape(b, h, vocab)


# speedup vs baseline: 1.4574x; 1.1075x over previous
import functools

import numpy as np
import jax
import jax.numpy as jnp
from jax.experimental import pallas as pl
from jax.experimental.pallas import tpu as pltpu

_ROWS = 8
_V = 32768
_SL = 1024
_NSL = _V // _SL
_L = np.float32(1.0 - 16 * 2.0**-24)
_NCAND = 128


def _st_softmax_block(x_ref, o_ref, idx_ref):
    x = x_ref[...]
    inf = jnp.float32(np.inf)

    macc = x[:, :_SL]
    for k in range(1, _NSL):
        macc = jnp.maximum(macc, x[:, k * _SL:(k + 1) * _SL])
    m = jnp.max(macc, axis=1, keepdims=True)

    base = jax.lax.broadcasted_iota(
        jnp.int32, (_ROWS, _SL), 1).astype(jnp.float32)
    sacc = None
    mn = jnp.full((_ROWS, _SL), inf, jnp.float32)
    mx = jnp.full((_ROWS, _SL), -inf, jnp.float32)
    for k in range(_NSL):
        u = jnp.exp(x[:, k * _SL:(k + 1) * _SL] - m)
        fio = base + jnp.float32(k * _SL)
        mask = u >= _L
        sacc = u if sacc is None else sacc + u
        mn = jnp.minimum(mn, jnp.where(mask, fio, inf))
        mx = jnp.maximum(mx, jnp.where(mask, fio, -inf))
    s = jnp.sum(sacc, axis=1, keepdims=True)
    mnr = jnp.min(mn, axis=1, keepdims=True)
    mxr = jnp.max(mx, axis=1, keepdims=True)

    idx_ref[...] = jnp.broadcast_to(mnr, (_ROWS, 128))

    @pl.when(jnp.any(mnr != mxr))
    def _exact_tie_fallback():
        umax = jnp.exp(jnp.zeros((_ROWS, 1), jnp.float32))
        pmax = umax / s
        k2 = jax.lax.broadcasted_iota(jnp.int32, (_ROWS, _NCAND), 1)
        ucand = jax.lax.bitcast_convert_type(
            jax.lax.bitcast_convert_type(umax, jnp.int32) - k2, jnp.float32)
        in_bucket = (ucand / s) == pmax
        c = jnp.min(jnp.where(in_bucket, ucand, inf), axis=1, keepdims=True)
        u = jnp.exp(x - m)
        fiota = jax.lax.broadcasted_iota(
            jnp.int32, (_ROWS, _V), 1).astype(jnp.float32)
        exact = jnp.min(jnp.where(u >= c, fiota, inf), axis=1, keepdims=True)
        idx_ref[...] = jnp.broadcast_to(exact, (_ROWS, 128))

    idx = idx_ref[:, :1].astype(jnp.int32)
    umax = jnp.exp(jnp.zeros((_ROWS, 1), jnp.float32))
    pmax = umax / s
    v = (1.0 - pmax) + pmax
    iota = jax.lax.broadcasted_iota(jnp.int32, (_ROWS, _V), 1)
    o_ref[...] = jnp.where(iota == idx, v, 0.0)


@jax.jit
def kernel(logits):
    b, h, vocab = logits.shape
    x = logits.reshape(b * h, vocab)
    out = pl.pallas_call(
        _st_softmax_block,
        grid=(b * h // _ROWS,),
        in_specs=[pl.BlockSpec((_ROWS, vocab), lambda i: (i, 0))],
        out_specs=pl.BlockSpec((_ROWS, vocab), lambda i: (i, 0)),
        out_shape=jax.ShapeDtypeStruct((b * h, vocab), jnp.float32),
        scratch_shapes=[pltpu.VMEM((_ROWS, 128), jnp.float32)],
    )(x)
    return out.reshape(b, h, vocab)
